# baseline (speedup 1.0000x reference)
import functools

import jax
import jax.numpy as jnp
from jax import lax
from jax.experimental import pallas as pl
from jax.experimental.pallas import tpu as pltpu
from jax.experimental.pallas import tpu_sc as plsc

_B = 16384
_D = 32
_K = 20
_NC = 2
_NS = 16
_L = 16
_NW = _NC * _NS
_PER_W = _B // _NW
_CHUNK = 64
_NCHUNK = _PER_W // _CHUNK
_NEG_ROWS = _CHUNK * _K
_IDXW = 128
_NEG_DMAS = _NEG_ROWS // _IDXW


def _softplus(z):
    a = jnp.abs(z)
    u = jnp.exp(-a)
    s = u / (u + 2.0)
    s2 = s * s
    p = jnp.float32(1.0 / 11.0)
    for c in (1.0 / 9.0, 1.0 / 7.0, 1.0 / 5.0, 1.0 / 3.0, 1.0):
        p = jnp.float32(c) + s2 * p
    return jnp.maximum(z, 0.0) + (2.0 * s) * p


def _clip(x):
    return jnp.minimum(jnp.maximum(x, -10.0), 10.0)


def _sc_loss(target_table, context_table, pos_target, pos_context, neg_idx2d):
    mesh = plsc.VectorSubcoreMesh(core_axis_name="c", subcore_axis_name="s")

    @functools.partial(
        pl.kernel,
        mesh=mesh,
        out_type=jax.ShapeDtypeStruct((_NW, _L), jnp.float32),
        compiler_params=pltpu.CompilerParams(
            needs_layout_passes=False, use_tc_tiling_on_sc=False
        ),
        scratch_types=[
            pltpu.VMEM((_CHUNK,), jnp.int32),
            pltpu.VMEM((_CHUNK,), jnp.int32),
            pltpu.VMEM((_NEG_ROWS,), jnp.int32),
            pltpu.VMEM((_CHUNK, _D), jnp.float32),
            pltpu.VMEM((_CHUNK, _D), jnp.float32),
            pltpu.VMEM((_NEG_ROWS, _D), jnp.float32),
            pltpu.VMEM((_L,), jnp.float32),
            pltpu.SemaphoreType.DMA,
        ],
    )
    def body(tt_hbm, ct_hbm, pt_hbm, pc_hbm, ni_hbm, out_hbm,
             tidx_v, cidx_v, negidx_v, t_v, c_v, neg_v, loss_v, sem):
        wid = lax.axis_index("s") * _NC + lax.axis_index("c")
        loss_v[...] = jnp.zeros((_L,), jnp.float32)

        def chunk_body(it, carry):
            base = wid * _PER_W + it * _CHUNK
            pltpu.sync_copy(pt_hbm.at[pl.ds(base, _CHUNK)], tidx_v)
            pltpu.sync_copy(pc_hbm.at[pl.ds(base, _CHUNK)], cidx_v)
            pltpu.sync_copy(ni_hbm.at[pl.ds(base * _K, _NEG_ROWS)], negidx_v)

            copies = [
                pltpu.async_copy(tt_hbm.at[tidx_v], t_v, sem),
                pltpu.async_copy(ct_hbm.at[cidx_v], c_v, sem),
            ]
            for j in range(_NEG_DMAS):
                copies.append(
                    pltpu.async_copy(
                        ct_hbm.at[negidx_v.at[pl.ds(j * _IDXW, _IDXW)]],
                        neg_v.at[pl.ds(j * _IDXW, _IDXW)],
                        sem,
                    )
                )
            for cp in copies:
                cp.wait()

            def group_body(g, gcarry):
                iot = lax.iota(jnp.int32, _L)
                rows = g * _L + iot
                rows_k = [rows * _K + k for k in range(_K)]
                acc = [jnp.zeros((_L,), jnp.float32) for _ in range(_K + 1)]
                for d in range(_D):
                    cold = jnp.full((_L,), d, jnp.int32)
                    tcol = plsc.load_gather(t_v, [rows, cold])
                    ccol = plsc.load_gather(c_v, [rows, cold])
                    acc[0] = acc[0] + tcol * ccol
                    for k in range(_K):
                        ncol = plsc.load_gather(neg_v, [rows_k[k], cold])
                        acc[k + 1] = acc[k + 1] + tcol * ncol
                total = _softplus(-_clip(acc[0]))
                for k in range(_K):
                    total = total + _softplus(_clip(acc[k + 1]))
                loss_v[...] = loss_v[...] + total
                return gcarry

            lax.fori_loop(0, _CHUNK // _L, group_body, 0)
            return carry

        lax.fori_loop(0, _NCHUNK, chunk_body, 0)
        pltpu.sync_copy(loss_v, out_hbm.at[wid])

    return body(target_table, context_table, pos_target, pos_context,
                neg_idx2d)


def kernel(target_table, context_table, pos_target, pos_context, neg_context):
    neg_flat = neg_context.reshape(_B * _K)
    partials = _sc_loss(target_table, context_table,
                        pos_target.astype(jnp.int32),
                        pos_context.astype(jnp.int32),
                        neg_flat.astype(jnp.int32))
    return jnp.sum(partials) / jnp.float32(_B)
